# baseline (speedup 1.0000x reference)
import functools

import jax
import jax.numpy as jnp
from jax import lax
from jax.experimental import pallas as pl
from jax.experimental.pallas import tpu as pltpu
from jax.experimental.pallas import tpu_sc as plsc

_VOCAB = 1000000
_EMB = 64
_HID = 64
_BATCH = 1024
_SEQ = 50
_NTOK = _BATCH * _SEQ
_GW = 128


_NW = 32
_CW = _BATCH // _NW


def _sc_gather(emb, inputs):
    mesh = plsc.VectorSubcoreMesh(core_axis_name="core",
                                  subcore_axis_name="subcore")
    cp = pltpu.CompilerParams()
    if "needs_layout_passes" in pltpu.CompilerParams.__dataclass_fields__:
        import dataclasses
        cp = dataclasses.replace(cp, needs_layout_passes=False)

    @functools.partial(
        pl.kernel,
        out_type=jax.ShapeDtypeStruct((_NTOK, _EMB), jnp.float32),
        mesh=mesh,
        compiler_params=cp,
        scratch_types=[
            pltpu.VMEM((_BATCH,), jnp.int32),
            pltpu.VMEM((_CW, _EMB), jnp.float32),
            pltpu.SemaphoreType.DMA,
        ],
    )
    def k(emb_hbm, i_hbm, o_hbm, idx_vmem, rows_v, sem):
        wid = lax.axis_index("subcore") * 2 + lax.axis_index("core")
        c0 = wid * _CW
        lane = lax.broadcasted_iota(jnp.int32, (16,), 0)

        @pl.loop(0, _SEQ)
        def _(t):
            pltpu.sync_copy(i_hbm.at[t], idx_vmem)

            for h in range(_CW // 16):
                vec = idx_vmem[pl.ds(c0 + h * 16, 16)]
                for kk in range(16):
                    v = jax.lax.reduce_max(
                        jnp.where(lane == kk, vec, 0), (0,))
                    pltpu.async_copy(emb_hbm.at[v], rows_v.at[h * 16 + kk],
                                     sem)

            @pl.loop(0, _CW)
            def _(j):
                pltpu.make_async_copy(emb_hbm.at[0], rows_v.at[j], sem).wait()

            pltpu.sync_copy(rows_v, o_hbm.at[pl.ds(t * _BATCH + c0, _CW)])

    return k(emb, _tc_transpose(inputs))


def _transpose_body(i_ref, o_ref):
    o_ref[...] = i_ref[...].T


_TBLK = 1024
_TGRID = -(-_VOCAB // _TBLK)


def _tc_table_transpose(embT):
    return pl.pallas_call(
        _transpose_body,
        grid=(_TGRID,),
        in_specs=[pl.BlockSpec((_EMB, _TBLK), lambda i: (0, i))],
        out_specs=pl.BlockSpec((_TBLK, _EMB), lambda i: (i, 0)),
        out_shape=jax.ShapeDtypeStruct((_VOCAB, _EMB), jnp.float32),
    )(embT)


def _tc_transpose(inputs):
    return pl.pallas_call(
        _transpose_body,
        out_shape=jax.ShapeDtypeStruct((_SEQ, _BATCH), jnp.int32),
    )(inputs)


def _lstm_body(x_ref, wf_ref, bf_ref, wib_ref, bb_ref, wcls_ref, bcls_ref,
               out_ref):
    wf = wf_ref[...]
    bf = bf_ref[...]

    def step(t, carry):
        h, c = carry
        xt = x_ref[pl.ds(t * _BATCH, _BATCH), :]
        xh = jnp.concatenate([xt, h], axis=1)
        gates = jnp.dot(xh, wf, preferred_element_type=jnp.float32) + bf
        i_, f_, g_, o_ = jnp.split(gates, 4, axis=1)
        c = jax.nn.sigmoid(f_) * c + jax.nn.sigmoid(i_) * jnp.tanh(g_)
        h = jax.nn.sigmoid(o_) * jnp.tanh(c)
        return (h, c)

    h0 = jnp.zeros((_BATCH, _HID), jnp.float32)
    h_f, _ = lax.fori_loop(0, _SEQ, step, (h0, h0))

    xt = x_ref[pl.ds((_SEQ - 1) * _BATCH, _BATCH), :]
    gb = jnp.dot(xt, wib_ref[...], preferred_element_type=jnp.float32) \
        + bb_ref[...]
    ib, _, gbb, ob = jnp.split(gb, 4, axis=1)
    cb = jax.nn.sigmoid(ib) * jnp.tanh(gbb)
    h_b = jax.nn.sigmoid(ob) * jnp.tanh(cb)

    last = jnp.concatenate([h_f, h_b], axis=1)
    out_ref[...] = jnp.dot(last, wcls_ref[...],
                           preferred_element_type=jnp.float32) + bcls_ref[...]


def _tc_lstm(x_tb, wf, bf, wib, bb, wcls, bcls, interpret=False):
    return pl.pallas_call(
        _lstm_body,
        out_shape=jax.ShapeDtypeStruct((_BATCH, 2), jnp.float32),
        interpret=interpret,
    )(x_tb, wf, bf, wib, bb, wcls, bcls)


def kernel(inputs, emb, W_ih_f, W_hh_f, b_ih_f, b_hh_f, W_ih_b, W_hh_b,
           b_ih_b, b_hh_b, W_cls, b_cls):
    x_tb = _sc_gather(_tc_table_transpose(emb.T), inputs)

    wf = jnp.concatenate([W_ih_f, W_hh_f], axis=1).T
    bf = (b_ih_f + b_hh_f).reshape(1, -1)
    wib = W_ih_b.T
    bb = (b_ih_b + b_hh_b).reshape(1, -1)
    wcls = W_cls.T
    bcls = b_cls.reshape(1, -1)
    return _tc_lstm(x_tb, wf, bf, wib, bb, wcls, bcls)


# speedup vs baseline: 1.9577x; 1.9577x over previous
import functools

import jax
import jax.numpy as jnp
from jax import lax
from jax.experimental import pallas as pl
from jax.experimental.pallas import tpu as pltpu
from jax.experimental.pallas import tpu_sc as plsc

_VOCAB = 1000000
_EMB = 64
_HID = 64
_BATCH = 1024
_SEQ = 50
_NTOK = _BATCH * _SEQ
_GW = 128


_NW = 32
_CW = _BATCH // _NW


def _sc_gather(emb, inputs):
    mesh = plsc.VectorSubcoreMesh(core_axis_name="core",
                                  subcore_axis_name="subcore")
    cp = pltpu.CompilerParams()
    if "needs_layout_passes" in pltpu.CompilerParams.__dataclass_fields__:
        import dataclasses
        cp = dataclasses.replace(cp, needs_layout_passes=False)

    @functools.partial(
        pl.kernel,
        out_type=jax.ShapeDtypeStruct((_NTOK, _EMB), jnp.float32),
        mesh=mesh,
        compiler_params=cp,
        scratch_types=[
            pltpu.VMEM((_SEQ, _BATCH), jnp.int32),
            pltpu.VMEM((2, _CW, _EMB), jnp.float32),
            pltpu.SemaphoreType.DMA,
            pltpu.SemaphoreType.DMA,
            pltpu.SemaphoreType.DMA,
            pltpu.SemaphoreType.DMA,
        ],
    )
    def k(emb_hbm, i_hbm, o_hbm, idx2d, rows2, gs0, gs1, os0, os1):
        wid = lax.axis_index("subcore") * 2 + lax.axis_index("core")
        c0 = wid * _CW
        lane = lax.broadcasted_iota(jnp.int32, (16,), 0)
        gsems = (gs0, gs1)
        osems = (os0, os1)

        pltpu.sync_copy(i_hbm, idx2d)

        def out_wait(par):
            pltpu.make_async_copy(rows2.at[par],
                                  o_hbm.at[pl.ds(0, _CW)],
                                  osems[par]).wait()

        def fire(t, par):
            for h in range(_CW // 16):
                vec = idx2d[t, pl.ds(c0 + h * 16, 16)]
                for kk in range(16):
                    v = jax.lax.reduce_max(
                        jnp.where(lane == kk, vec, 0), (0,))
                    pltpu.async_copy(emb_hbm.at[v],
                                     rows2.at[par, h * 16 + kk], gsems[par])

        def drain(par):
            @pl.loop(0, _CW)
            def _(j):
                pltpu.make_async_copy(emb_hbm.at[0], rows2.at[par, j],
                                      gsems[par]).wait()

        def out_start(t, par):
            pltpu.async_copy(rows2.at[par],
                             o_hbm.at[pl.ds(t * _BATCH + c0, _CW)],
                             osems[par])

        fire(0, 0)

        @pl.loop(0, _SEQ // 2)
        def _(p):
            t0 = 2 * p

            @pl.when(p > 0)
            def _():
                out_wait(1)
            fire(t0 + 1, 1)
            drain(0)
            out_start(t0, 0)

            @pl.when(p < _SEQ // 2 - 1)
            def _():
                out_wait(0)
                fire(t0 + 2, 0)
            drain(1)
            out_start(t0 + 1, 1)

        out_wait(0)
        out_wait(1)

    return k(emb, _tc_transpose(inputs))


def _transpose_body(i_ref, o_ref):
    o_ref[...] = i_ref[...].T


_TBLK = 1024
_TGRID = -(-_VOCAB // _TBLK)


def _tc_table_transpose(embT):
    return pl.pallas_call(
        _transpose_body,
        grid=(_TGRID,),
        in_specs=[pl.BlockSpec((_EMB, _TBLK), lambda i: (0, i))],
        out_specs=pl.BlockSpec((_TBLK, _EMB), lambda i: (i, 0)),
        out_shape=jax.ShapeDtypeStruct((_VOCAB, _EMB), jnp.float32),
    )(embT)


def _tc_transpose(inputs):
    return pl.pallas_call(
        _transpose_body,
        out_shape=jax.ShapeDtypeStruct((_SEQ, _BATCH), jnp.int32),
    )(inputs)


def _lstm_body(x_ref, wf_ref, bf_ref, wib_ref, bb_ref, wcls_ref, bcls_ref,
               out_ref):
    wf = wf_ref[...]
    bf = bf_ref[...]

    def step(t, carry):
        h, c = carry
        xt = x_ref[pl.ds(t * _BATCH, _BATCH), :]
        xh = jnp.concatenate([xt, h], axis=1)
        gates = jnp.dot(xh, wf, preferred_element_type=jnp.float32) + bf
        i_, f_, g_, o_ = jnp.split(gates, 4, axis=1)
        c = jax.nn.sigmoid(f_) * c + jax.nn.sigmoid(i_) * jnp.tanh(g_)
        h = jax.nn.sigmoid(o_) * jnp.tanh(c)
        return (h, c)

    h0 = jnp.zeros((_BATCH, _HID), jnp.float32)
    h_f, _ = lax.fori_loop(0, _SEQ, step, (h0, h0))

    xt = x_ref[pl.ds((_SEQ - 1) * _BATCH, _BATCH), :]
    gb = jnp.dot(xt, wib_ref[...], preferred_element_type=jnp.float32) \
        + bb_ref[...]
    ib, _, gbb, ob = jnp.split(gb, 4, axis=1)
    cb = jax.nn.sigmoid(ib) * jnp.tanh(gbb)
    h_b = jax.nn.sigmoid(ob) * jnp.tanh(cb)

    last = jnp.concatenate([h_f, h_b], axis=1)
    out_ref[...] = jnp.dot(last, wcls_ref[...],
                           preferred_element_type=jnp.float32) + bcls_ref[...]


def _tc_lstm(x_tb, wf, bf, wib, bb, wcls, bcls, interpret=False):
    return pl.pallas_call(
        _lstm_body,
        out_shape=jax.ShapeDtypeStruct((_BATCH, 2), jnp.float32),
        interpret=interpret,
    )(x_tb, wf, bf, wib, bb, wcls, bcls)


def kernel(inputs, emb, W_ih_f, W_hh_f, b_ih_f, b_hh_f, W_ih_b, W_hh_b,
           b_ih_b, b_hh_b, W_cls, b_cls):
    x_tb = _sc_gather(emb, inputs)

    wf = jnp.concatenate([W_ih_f, W_hh_f], axis=1).T
    bf = (b_ih_f + b_hh_f).reshape(1, -1)
    wib = W_ih_b.T
    bb = (b_ih_b + b_hh_b).reshape(1, -1)
    wcls = W_cls.T
    bcls = b_cls.reshape(1, -1)
    return _tc_lstm(x_tb, wf, bf, wib, bb, wcls, bcls)


# speedup vs baseline: 1.9933x; 1.0182x over previous
import functools

import jax
import jax.numpy as jnp
from jax import lax
from jax.experimental import pallas as pl
from jax.experimental.pallas import tpu as pltpu
from jax.experimental.pallas import tpu_sc as plsc

_VOCAB = 1000000
_EMB = 64
_HID = 64
_BATCH = 1024
_SEQ = 50
_NTOK = _BATCH * _SEQ
_GW = 128


_NW = 32
_CW = _BATCH // _NW


def _sc_gather(emb, inputs):
    mesh = plsc.VectorSubcoreMesh(core_axis_name="core",
                                  subcore_axis_name="subcore")
    cp = pltpu.CompilerParams()
    if "needs_layout_passes" in pltpu.CompilerParams.__dataclass_fields__:
        import dataclasses
        cp = dataclasses.replace(cp, needs_layout_passes=False)

    @functools.partial(
        pl.kernel,
        out_type=jax.ShapeDtypeStruct((_NTOK, _EMB), jnp.float32),
        mesh=mesh,
        compiler_params=cp,
        scratch_types=[
            pltpu.VMEM((_SEQ, _BATCH), jnp.int32),
            pltpu.VMEM((2, _CW, _EMB), jnp.float32),
            pltpu.SemaphoreType.DMA,
            pltpu.SemaphoreType.DMA,
            pltpu.SemaphoreType.DMA,
            pltpu.SemaphoreType.DMA,
        ],
    )
    def k(emb_hbm, i_hbm, o_hbm, idx2d, rows2, gs0, gs1, os0, os1):
        wid = lax.axis_index("subcore") * 2 + lax.axis_index("core")
        c0 = wid * _CW
        lane = lax.broadcasted_iota(jnp.int32, (16,), 0)
        gsems = (gs0, gs1)
        osems = (os0, os1)

        pltpu.sync_copy(i_hbm, idx2d)

        def out_wait(par):
            pltpu.make_async_copy(rows2.at[par],
                                  o_hbm.at[pl.ds(0, _CW)],
                                  osems[par]).wait()

        def fire(t, par):
            for h in range(_CW // 16):
                vec = idx2d[t, pl.ds(c0 + h * 16, 16)]
                for kk in range(16):
                    v = jax.lax.reduce_max(
                        jnp.where(lane == kk, vec, 0), (0,))
                    pltpu.async_copy(emb_hbm.at[v],
                                     rows2.at[par, h * 16 + kk], gsems[par])

        def drain(par):
            @pl.loop(0, _CW)
            def _(j):
                pltpu.make_async_copy(emb_hbm.at[0], rows2.at[par, j],
                                      gsems[par]).wait()

        def out_start(t, par):
            pltpu.async_copy(rows2.at[par],
                             o_hbm.at[pl.ds(t * _BATCH + c0, _CW)],
                             osems[par])

        fire(0, 0)

        @pl.loop(0, _SEQ // 2)
        def _(p):
            t0 = 2 * p

            @pl.when(p > 0)
            def _():
                out_wait(1)
            fire(t0 + 1, 1)
            drain(0)
            out_start(t0, 0)

            @pl.when(p < _SEQ // 2 - 1)
            def _():
                out_wait(0)
                fire(t0 + 2, 0)
            drain(1)
            out_start(t0 + 1, 1)

        out_wait(0)
        out_wait(1)

    return k(emb, _tc_transpose(inputs))


def _transpose_body(i_ref, o_ref):
    o_ref[...] = i_ref[...].T


_TBLK = 1024
_TGRID = -(-_VOCAB // _TBLK)


def _tc_table_transpose(embT):
    return pl.pallas_call(
        _transpose_body,
        grid=(_TGRID,),
        in_specs=[pl.BlockSpec((_EMB, _TBLK), lambda i: (0, i))],
        out_specs=pl.BlockSpec((_TBLK, _EMB), lambda i: (i, 0)),
        out_shape=jax.ShapeDtypeStruct((_VOCAB, _EMB), jnp.float32),
    )(embT)


def _tc_transpose(inputs):
    return pl.pallas_call(
        _transpose_body,
        out_shape=jax.ShapeDtypeStruct((_SEQ, _BATCH), jnp.int32),
    )(inputs)


def _lstm_body(x_ref, wf_ref, bf_ref, wib_ref, bb_ref, wcls_ref, bcls_ref,
               out_ref):
    wf = wf_ref[...]
    bf = bf_ref[...]

    def step(t, carry):
        h, c = carry
        xt = x_ref[pl.ds(t * _BATCH, _BATCH), :]
        xh = jnp.concatenate([xt, h], axis=1)
        gates = jnp.dot(xh, wf, preferred_element_type=jnp.float32) + bf
        i_, f_, g_, o_ = jnp.split(gates, 4, axis=1)
        c = jax.nn.sigmoid(f_) * c + jax.nn.sigmoid(i_) * jnp.tanh(g_)
        h = jax.nn.sigmoid(o_) * jnp.tanh(c)
        return (h, c)

    h0 = jnp.zeros((_BATCH, _HID), jnp.float32)
    h_f, _ = lax.fori_loop(0, _SEQ, step, (h0, h0), unroll=2)

    xt = x_ref[pl.ds((_SEQ - 1) * _BATCH, _BATCH), :]
    gb = jnp.dot(xt, wib_ref[...], preferred_element_type=jnp.float32) \
        + bb_ref[...]
    ib, _, gbb, ob = jnp.split(gb, 4, axis=1)
    cb = jax.nn.sigmoid(ib) * jnp.tanh(gbb)
    h_b = jax.nn.sigmoid(ob) * jnp.tanh(cb)

    last = jnp.concatenate([h_f, h_b], axis=1)
    out_ref[...] = jnp.dot(last, wcls_ref[...],
                           preferred_element_type=jnp.float32) + bcls_ref[...]


def _tc_lstm(x_tb, wf, bf, wib, bb, wcls, bcls, interpret=False):
    return pl.pallas_call(
        _lstm_body,
        out_shape=jax.ShapeDtypeStruct((_BATCH, 2), jnp.float32),
        interpret=interpret,
    )(x_tb, wf, bf, wib, bb, wcls, bcls)


def kernel(inputs, emb, W_ih_f, W_hh_f, b_ih_f, b_hh_f, W_ih_b, W_hh_b,
           b_ih_b, b_hh_b, W_cls, b_cls):
    x_tb = _sc_gather(emb, inputs)

    wf = jnp.concatenate([W_ih_f, W_hh_f], axis=1).T
    bf = (b_ih_f + b_hh_f).reshape(1, -1)
    wib = W_ih_b.T
    bb = (b_ih_b + b_hh_b).reshape(1, -1)
    wcls = W_cls.T
    bcls = b_cls.reshape(1, -1)
    return _tc_lstm(x_tb, wf, bf, wib, bb, wcls, bcls)


# speedup vs baseline: 2.0213x; 1.0141x over previous
import functools

import jax
import jax.numpy as jnp
from jax import lax
from jax.experimental import pallas as pl
from jax.experimental.pallas import tpu as pltpu
from jax.experimental.pallas import tpu_sc as plsc

_VOCAB = 1000000
_EMB = 64
_HID = 64
_BATCH = 1024
_SEQ = 50
_NTOK = _BATCH * _SEQ
_GW = 128


_NW = 32
_CW = _BATCH // _NW


def _sc_gather(emb, inputs):
    mesh = plsc.VectorSubcoreMesh(core_axis_name="core",
                                  subcore_axis_name="subcore")
    cp = pltpu.CompilerParams()
    if "needs_layout_passes" in pltpu.CompilerParams.__dataclass_fields__:
        import dataclasses
        cp = dataclasses.replace(cp, needs_layout_passes=False)

    @functools.partial(
        pl.kernel,
        out_type=jax.ShapeDtypeStruct((_NTOK, _EMB), jnp.float32),
        mesh=mesh,
        compiler_params=cp,
        scratch_types=[
            pltpu.VMEM((_SEQ, _BATCH), jnp.int32),
            pltpu.VMEM((2, _CW, _EMB), jnp.float32),
            pltpu.SemaphoreType.DMA,
            pltpu.SemaphoreType.DMA,
            pltpu.SemaphoreType.DMA,
            pltpu.SemaphoreType.DMA,
        ],
    )
    def k(emb_hbm, i_hbm, o_hbm, idx2d, rows2, gs0, gs1, os0, os1):
        wid = lax.axis_index("subcore") * 2 + lax.axis_index("core")
        c0 = wid * _CW
        lane = lax.broadcasted_iota(jnp.int32, (16,), 0)
        gsems = (gs0, gs1)
        osems = (os0, os1)

        pltpu.sync_copy(i_hbm, idx2d)

        def out_wait(par):
            pltpu.make_async_copy(rows2.at[par],
                                  o_hbm.at[pl.ds(0, _CW)],
                                  osems[par]).wait()

        def fire(t, par):
            for h in range(_CW // 16):
                vec = idx2d[t, pl.ds(c0 + h * 16, 16)]
                for kk in range(16):
                    v = jax.lax.reduce_max(
                        jnp.where(lane == kk, vec, 0), (0,))
                    pltpu.async_copy(emb_hbm.at[v],
                                     rows2.at[par, h * 16 + kk], gsems[par])

        def drain(par):
            @pl.loop(0, _CW)
            def _(j):
                pltpu.make_async_copy(emb_hbm.at[0], rows2.at[par, j],
                                      gsems[par]).wait()

        def out_start(t, par):
            pltpu.async_copy(rows2.at[par],
                             o_hbm.at[pl.ds(t * _BATCH + c0, _CW)],
                             osems[par])

        fire(0, 0)

        @pl.loop(0, _SEQ // 2)
        def _(p):
            t0 = 2 * p

            @pl.when(p > 0)
            def _():
                out_wait(1)
            fire(t0 + 1, 1)
            drain(0)
            out_start(t0, 0)

            @pl.when(p < _SEQ // 2 - 1)
            def _():
                out_wait(0)
                fire(t0 + 2, 0)
            drain(1)
            out_start(t0 + 1, 1)

        out_wait(0)
        out_wait(1)

    return k(emb, _tc_transpose(inputs))


def _transpose_body(i_ref, o_ref):
    o_ref[...] = i_ref[...].T


_TBLK = 1024
_TGRID = -(-_VOCAB // _TBLK)


def _tc_table_transpose(embT):
    return pl.pallas_call(
        _transpose_body,
        grid=(_TGRID,),
        in_specs=[pl.BlockSpec((_EMB, _TBLK), lambda i: (0, i))],
        out_specs=pl.BlockSpec((_TBLK, _EMB), lambda i: (i, 0)),
        out_shape=jax.ShapeDtypeStruct((_VOCAB, _EMB), jnp.float32),
    )(embT)


def _tc_transpose(inputs):
    return pl.pallas_call(
        _transpose_body,
        out_shape=jax.ShapeDtypeStruct((_SEQ, _BATCH), jnp.int32),
    )(inputs)


def _lstm_body(x_ref, wf_ref, bf_ref, wib_ref, bb_ref, wcls_ref, bcls_ref,
               out_ref):
    wf = wf_ref[...]
    bf = bf_ref[...]

    def step(t, carry):
        h, c = carry
        xt = x_ref[pl.ds(t * _BATCH, _BATCH), :]
        xh = jnp.concatenate([xt, h], axis=1)
        gates = jnp.dot(xh, wf, preferred_element_type=jnp.float32) + bf
        i_, f_, g_, o_ = jnp.split(gates, 4, axis=1)
        c = jax.nn.sigmoid(f_) * c + jax.nn.sigmoid(i_) * jnp.tanh(g_)
        h = jax.nn.sigmoid(o_) * jnp.tanh(c)
        return (h, c)

    h0 = jnp.zeros((_BATCH, _HID), jnp.float32)
    h_f, _ = lax.fori_loop(0, _SEQ, step, (h0, h0), unroll=5)

    xt = x_ref[pl.ds((_SEQ - 1) * _BATCH, _BATCH), :]
    gb = jnp.dot(xt, wib_ref[...], preferred_element_type=jnp.float32) \
        + bb_ref[...]
    ib, _, gbb, ob = jnp.split(gb, 4, axis=1)
    cb = jax.nn.sigmoid(ib) * jnp.tanh(gbb)
    h_b = jax.nn.sigmoid(ob) * jnp.tanh(cb)

    last = jnp.concatenate([h_f, h_b], axis=1)
    out_ref[...] = jnp.dot(last, wcls_ref[...],
                           preferred_element_type=jnp.float32) + bcls_ref[...]


def _tc_lstm(x_tb, wf, bf, wib, bb, wcls, bcls, interpret=False):
    return pl.pallas_call(
        _lstm_body,
        out_shape=jax.ShapeDtypeStruct((_BATCH, 2), jnp.float32),
        interpret=interpret,
    )(x_tb, wf, bf, wib, bb, wcls, bcls)


def kernel(inputs, emb, W_ih_f, W_hh_f, b_ih_f, b_hh_f, W_ih_b, W_hh_b,
           b_ih_b, b_hh_b, W_cls, b_cls):
    x_tb = _sc_gather(emb, inputs)

    wf = jnp.concatenate([W_ih_f, W_hh_f], axis=1).T
    bf = (b_ih_f + b_hh_f).reshape(1, -1)
    wib = W_ih_b.T
    bb = (b_ih_b + b_hh_b).reshape(1, -1)
    wcls = W_cls.T
    bcls = b_cls.reshape(1, -1)
    return _tc_lstm(x_tb, wf, bf, wib, bb, wcls, bcls)
